# baseline (speedup 1.0000x reference)
import jax
import jax.numpy as jnp
from jax.experimental import pallas as pl
from jax.experimental.pallas import tpu as pltpu

TOP_K = 16
GB = 16
LT = 1024


def _mem_kernel(feat_ref, keys_ref, mem_ref, out_ref, w_ref):
    @pl.when((pl.program_id(0) == 0) & (pl.program_id(1) == 0))
    def _compute_weights():
        f = feat_ref[...]
        k = keys_ref[:, 0, :]
        dots = jax.lax.dot_general(
            f, k, (((1,), (1,)), ((), ())),
            preferred_element_type=jnp.float32)
        qn = jnp.sqrt(jnp.sum(f * f, axis=1, keepdims=True))
        kn = jnp.sqrt(jnp.sum(k * k, axis=1, keepdims=True))
        denom = jnp.maximum(qn * kn.T, 1e-8)
        sim = dots / denom

        B, M = sim.shape
        col = jax.lax.broadcasted_iota(jnp.int32, (B, M), 1)
        masked = sim
        sel = jnp.zeros((B, M), dtype=jnp.bool_)
        neg_inf = jnp.float32(-jnp.inf)
        for _ in range(TOP_K):
            idx = jnp.argmax(masked, axis=1)
            onehot = col == idx[:, None]
            sel = jnp.logical_or(sel, onehot)
            masked = jnp.where(onehot, neg_inf, masked)
        w = jnp.where(sel, jnp.maximum(sim, 0.0), 0.0)
        w = w / jnp.sum(w, axis=1, keepdims=True)
        w_ref[...] = w

    w = w_ref[...]
    for g in range(GB):
        out_ref[:, g, :] = jax.lax.dot_general(
            w, mem_ref[:, g, :], (((1,), (0,)), ((), ())),
            preferred_element_type=jnp.float32)


@jax.jit
def kernel(features_, keys, memory):
    B, D = features_.shape
    M = keys.shape[0]
    G, L = memory.shape[1], memory.shape[2]

    grid = (G // GB, L // LT)
    ctx = pl.pallas_call(
        _mem_kernel,
        grid=grid,
        in_specs=[
            pl.BlockSpec((B, D), lambda i, j: (0, 0)),
            pl.BlockSpec((M, 1, D), lambda i, j: (0, 0, 0)),
            pl.BlockSpec((M, GB, LT), lambda i, j: (0, i, j)),
        ],
        out_specs=pl.BlockSpec((B, GB, LT), lambda i, j: (0, i, j)),
        out_shape=jax.ShapeDtypeStruct((B, G, L), jnp.float32),
        scratch_shapes=[pltpu.VMEM((B, M), jnp.float32)],
    )(features_, keys, memory)
    return ctx


# speedup vs baseline: 1.0265x; 1.0265x over previous
import jax
import jax.numpy as jnp
from jax.experimental import pallas as pl
from jax.experimental.pallas import tpu as pltpu

TOP_K = 16
GB = 64
LT = 256


def _mem_kernel(feat_ref, keys_ref, mem_ref, out_ref, w_ref):
    @pl.when((pl.program_id(0) == 0) & (pl.program_id(1) == 0))
    def _compute_weights():
        f = feat_ref[...]
        k = keys_ref[:, 0, :]
        dots = jax.lax.dot_general(
            f, k, (((1,), (1,)), ((), ())),
            preferred_element_type=jnp.float32)
        qn = jnp.sqrt(jnp.sum(f * f, axis=1, keepdims=True))
        kn = jnp.sqrt(jnp.sum(k * k, axis=1, keepdims=True))
        denom = jnp.maximum(qn * kn.T, 1e-8)
        sim = dots / denom

        B, M = sim.shape
        col = jax.lax.broadcasted_iota(jnp.int32, (B, M), 1)
        masked = sim
        sel = jnp.zeros((B, M), dtype=jnp.bool_)
        neg_inf = jnp.float32(-jnp.inf)
        for _ in range(TOP_K):
            idx = jnp.argmax(masked, axis=1)
            onehot = col == idx[:, None]
            sel = jnp.logical_or(sel, onehot)
            masked = jnp.where(onehot, neg_inf, masked)
        w = jnp.where(sel, jnp.maximum(sim, 0.0), 0.0)
        w = w / jnp.sum(w, axis=1, keepdims=True)
        w_ref[...] = w

    w = w_ref[...]
    for g in range(GB):
        out_ref[:, g, :] = jax.lax.dot_general(
            w, mem_ref[:, g, :], (((1,), (0,)), ((), ())),
            preferred_element_type=jnp.float32)


@jax.jit
def kernel(features_, keys, memory):
    B, D = features_.shape
    M = keys.shape[0]
    G, L = memory.shape[1], memory.shape[2]

    grid = (G // GB, L // LT)
    ctx = pl.pallas_call(
        _mem_kernel,
        grid=grid,
        in_specs=[
            pl.BlockSpec((B, D), lambda i, j: (0, 0)),
            pl.BlockSpec((M, 1, D), lambda i, j: (0, 0, 0)),
            pl.BlockSpec((M, GB, LT), lambda i, j: (0, i, j)),
        ],
        out_specs=pl.BlockSpec((B, GB, LT), lambda i, j: (0, i, j)),
        out_shape=jax.ShapeDtypeStruct((B, G, L), jnp.float32),
        scratch_shapes=[pltpu.VMEM((B, M), jnp.float32)],
    )(features_, keys, memory)
    return ctx


# speedup vs baseline: 1.1591x; 1.1292x over previous
import jax
import jax.numpy as jnp
from jax.experimental import pallas as pl
from jax.experimental.pallas import tpu as pltpu

TOP_K = 16
GB = 32
LT = 512


def _mem_kernel(feat_ref, keys_ref, mem_ref, out_ref, w_ref):
    @pl.when((pl.program_id(0) == 0) & (pl.program_id(1) == 0))
    def _compute_weights():
        f = feat_ref[...]
        k = keys_ref[:, 0, :]
        dots = jax.lax.dot_general(
            f, k, (((1,), (1,)), ((), ())),
            preferred_element_type=jnp.float32)
        qn = jnp.sqrt(jnp.sum(f * f, axis=1, keepdims=True))
        kn = jnp.sqrt(jnp.sum(k * k, axis=1, keepdims=True))
        denom = jnp.maximum(qn * kn.T, 1e-8)
        sim = dots / denom

        B, M = sim.shape
        col = jax.lax.broadcasted_iota(jnp.int32, (B, M), 1)
        masked = sim
        sel = jnp.zeros((B, M), dtype=jnp.bool_)
        neg_inf = jnp.float32(-jnp.inf)
        for _ in range(TOP_K):
            idx = jnp.argmax(masked, axis=1)
            onehot = col == idx[:, None]
            sel = jnp.logical_or(sel, onehot)
            masked = jnp.where(onehot, neg_inf, masked)
        w = jnp.where(sel, jnp.maximum(sim, 0.0), 0.0)
        w = w / jnp.sum(w, axis=1, keepdims=True)
        w_ref[...] = w

    w = w_ref[...]
    for g in range(GB):
        out_ref[:, g, :] = jax.lax.dot_general(
            w, mem_ref[:, g, :], (((1,), (0,)), ((), ())),
            preferred_element_type=jnp.float32)


@jax.jit
def kernel(features_, keys, memory):
    B, D = features_.shape
    M = keys.shape[0]
    G, L = memory.shape[1], memory.shape[2]

    grid = (G // GB, L // LT)
    ctx = pl.pallas_call(
        _mem_kernel,
        grid=grid,
        in_specs=[
            pl.BlockSpec((B, D), lambda i, j: (0, 0)),
            pl.BlockSpec((M, 1, D), lambda i, j: (0, 0, 0)),
            pl.BlockSpec((M, GB, LT), lambda i, j: (0, i, j)),
        ],
        out_specs=pl.BlockSpec((B, GB, LT), lambda i, j: (0, i, j)),
        out_shape=jax.ShapeDtypeStruct((B, G, L), jnp.float32),
        scratch_shapes=[pltpu.VMEM((B, M), jnp.float32)],
    )(features_, keys, memory)
    return ctx
